# baseline (speedup 1.0000x reference)
import functools

import jax
import jax.numpy as jnp
from jax import lax
from jax.experimental import pallas as pl
from jax.experimental.pallas import tpu as pltpu
from jax.experimental.pallas import tpu_sc as plsc

NUM_CORES = 2
NUM_SUBCORES = 16
LANES = 16
NW = NUM_CORES * NUM_SUBCORES

PAIRS = 768
ROWS = PAIRS // NW
DIM = 12
PERIODS = 288
PT_WORDS = PERIODS * DIM
WT_WORDS = 7 * DIM
X_OFF = 0
TE_OFF = 2 * PAIRS
PT_OFF = TE_OFF + PAIRS * DIM
WT_OFF = PT_OFF + PT_WORDS
P_WORDS = WT_OFF + WT_WORDS


def _sc_body(pk, out, xbuf, obuf, tebuf, ptbuf, wtbuf, outbuf, sem):
    wid = lax.axis_index("s") * NUM_CORES + lax.axis_index("c")
    base = wid * ROWS

    c1 = pltpu.async_copy(pk.at[pl.ds(2 * base, 2 * ROWS)],
                          xbuf.at[pl.ds(0, 2 * ROWS)], sem)
    c2 = pltpu.async_copy(pk.at[pl.ds(TE_OFF + base * DIM, ROWS * DIM)],
                          tebuf.at[pl.ds(0, ROWS * DIM)], sem)
    c3 = pltpu.async_copy(pk.at[pl.ds(PT_OFF, PT_WORDS)],
                          ptbuf.at[pl.ds(0, PT_WORDS)], sem)
    c4 = pltpu.async_copy(pk.at[pl.ds(WT_OFF, WT_WORDS)],
                          wtbuf.at[pl.ds(0, WT_WORDS)], sem)
    c1.wait()

    par = lax.iota(jnp.int32, LANES) % 2
    scale = jnp.where(par == 0, float(PERIODS), 1.0)
    hi = jnp.where(par == 0, PERIODS - 1, 6)
    for k in range(2 * ROWS // LANES):
        v = xbuf[pl.ds(k * LANES, LANES)] * scale
        i = v.astype(jnp.int32)
        i = jnp.where(i.astype(jnp.float32) > v, i - 1, i)
        obuf[pl.ds(k * LANES, LANES)] = jnp.clip(i, 0, hi) * DIM

    c2.wait()
    c3.wait()
    c4.wait()

    def body(r, carry):
        p12 = obuf[pl.ds(2 * r, LANES)][0]
        w12 = obuf[pl.ds(2 * r + 1, LANES)][0]
        tev = tebuf[pl.ds(r * DIM, LANES)]
        pe = ptbuf[pl.ds(p12, LANES)]
        we = wtbuf[pl.ds(w12, LANES)]
        outbuf[pl.ds(r * DIM, LANES)] = tev * pe * we
        return carry

    lax.fori_loop(0, ROWS, body, 0)

    pltpu.sync_copy(outbuf.at[pl.ds(0, ROWS * DIM)],
                    out.at[pl.ds(base * DIM, ROWS * DIM)])


_sc_encoder = functools.partial(
    pl.kernel,
    mesh=plsc.VectorSubcoreMesh(core_axis_name="c", subcore_axis_name="s"),
    out_type=jax.ShapeDtypeStruct((PAIRS * DIM,), jnp.float32),
    scratch_types=[
        pltpu.VMEM((2 * ROWS + LANES,), jnp.float32),
        pltpu.VMEM((2 * ROWS + LANES,), jnp.int32),
        pltpu.VMEM((ROWS * DIM + LANES,), jnp.float32),
        pltpu.VMEM((PT_WORDS + LANES,), jnp.float32),
        pltpu.VMEM((WT_WORDS + LANES,), jnp.float32),
        pltpu.VMEM((ROWS * DIM + LANES,), jnp.float32),
        pltpu.SemaphoreType.DMA,
    ],
)(_sc_body)


def kernel(x, periods_table, weekend_table, node_embeddings, time_embeddings):
    b, t = x.shape[0], x.shape[1]
    pk = jnp.concatenate([
        x[:, :, 0, 1:3].reshape(2 * b * t),
        time_embeddings[:b].reshape(b * t * DIM),
        periods_table.reshape(PT_WORDS),
        weekend_table.reshape(WT_WORDS),
    ])
    out = _sc_encoder(pk)
    return node_embeddings, out.reshape(b, t, DIM)
